# baseline (speedup 1.0000x reference)
import functools

import jax
import jax.numpy as jnp
from jax.experimental import pallas as pl
from jax.experimental.pallas import tpu as pltpu

E = 8
TOPK = 2
D = 1024
FF = 512
T = 2048
CHUNK = 256


def _routing_col(logits, e):
    probs = jax.nn.softmax(logits.astype(jnp.float32), axis=-1)
    lane = jax.lax.broadcasted_iota(jnp.int32, probs.shape, 1)
    m1 = jnp.max(probs, axis=-1, keepdims=True)
    i1 = jnp.min(jnp.where(probs == m1, lane, E), axis=-1, keepdims=True)
    probs2 = jnp.where(lane == i1, -jnp.inf, probs)
    m2 = jnp.max(probs2, axis=-1, keepdims=True)
    i2 = jnp.min(jnp.where(probs2 == m2, lane, E), axis=-1, keepdims=True)
    s = m1 + m2
    w1 = m1 / s
    w2 = m2 / s
    return jnp.where(i1 == e, w1, 0.0) + jnp.where(i2 == e, w2, 0.0)


def _moe_body(logits_ref, x_ref, wu_ref, wg_ref, wd_ref, out_ref):
    e = pl.program_id(0)

    wu = wu_ref[0]
    wg = wg_ref[0]
    wd = wd_ref[0]

    dn = (((1,), (1,)), ((), ()))

    def chunk(i, _):
        sl = pl.ds(i * CHUNK, CHUNK)
        xc = x_ref[sl, :]
        w = _routing_col(logits_ref[sl, :], e)
        up = jax.lax.dot_general(xc, wu, dn, preferred_element_type=jnp.float32)
        gate = jax.lax.dot_general(xc, wg, dn, preferred_element_type=jnp.float32)
        gw = gate * w
        h = gw * jax.nn.sigmoid(gw) * (up * w)
        contrib = jax.lax.dot_general(
            h.astype(jnp.bfloat16), wd, dn, preferred_element_type=jnp.float32
        ) * w

        @pl.when(e == 0)
        def _():
            out_ref[sl, :] = contrib

        @pl.when(e != 0)
        def _():
            out_ref[sl, :] += contrib

        return 0

    jax.lax.fori_loop(0, T // CHUNK, chunk, 0, unroll=False)


@jax.jit
def kernel(x, router_logits_up, router_logits_gate, router_logits_down, w_up, w_gate, w_down):
    del router_logits_gate, router_logits_down
    x = x.astype(jnp.bfloat16)
    w_up = w_up.astype(jnp.bfloat16)
    w_gate = w_gate.astype(jnp.bfloat16)
    w_down = w_down.astype(jnp.bfloat16)
    return pl.pallas_call(
        _moe_body,
        grid=(E,),
        in_specs=[
            pl.BlockSpec((T, E), lambda e: (0, 0)),
            pl.BlockSpec((T, D), lambda e: (0, 0)),
            pl.BlockSpec((1, FF, D), lambda e: (e, 0, 0)),
            pl.BlockSpec((1, FF, D), lambda e: (e, 0, 0)),
            pl.BlockSpec((1, D, FF), lambda e: (e, 0, 0)),
        ],
        out_specs=pl.BlockSpec((T, D), lambda e: (0, 0)),
        out_shape=jax.ShapeDtypeStruct((T, D), jnp.float32),
    )(router_logits_up, x, w_up, w_gate, w_down)


# speedup vs baseline: 1.2640x; 1.2640x over previous
import functools

import jax
import jax.numpy as jnp
from jax.experimental import pallas as pl
from jax.experimental.pallas import tpu as pltpu

E = 8
TOPK = 2
D = 1024
FF = 512
T = 2048
CHUNK = 256


def _routing_col(logits, e):
    probs = jax.nn.softmax(logits.astype(jnp.float32), axis=-1)
    lane = jax.lax.broadcasted_iota(jnp.int32, probs.shape, 1)
    m1 = jnp.max(probs, axis=-1, keepdims=True)
    i1 = jnp.min(jnp.where(probs == m1, lane, E), axis=-1, keepdims=True)
    probs2 = jnp.where(lane == i1, -jnp.inf, probs)
    m2 = jnp.max(probs2, axis=-1, keepdims=True)
    i2 = jnp.min(jnp.where(probs2 == m2, lane, E), axis=-1, keepdims=True)
    s = m1 + m2
    w1 = m1 / s
    w2 = m2 / s
    return jnp.where(i1 == e, w1, 0.0) + jnp.where(i2 == e, w2, 0.0)


def _moe_body(logits_ref, x_ref, wu_ref, wg_ref, wd_ref, out_ref):
    e = pl.program_id(0)

    wu = wu_ref[0].astype(jnp.bfloat16)
    wg = wg_ref[0].astype(jnp.bfloat16)
    wd = wd_ref[0].astype(jnp.bfloat16)

    dn = (((1,), (1,)), ((), ()))

    def chunk(i, _):
        sl = pl.ds(i * CHUNK, CHUNK)
        xc = x_ref[sl, :].astype(jnp.bfloat16)
        w = _routing_col(logits_ref[sl, :], e)
        up = jax.lax.dot_general(xc, wu, dn, preferred_element_type=jnp.float32)
        gate = jax.lax.dot_general(xc, wg, dn, preferred_element_type=jnp.float32)
        gw = gate * w
        h = gw * jax.nn.sigmoid(gw) * (up * w)
        contrib = jax.lax.dot_general(
            h.astype(jnp.bfloat16), wd, dn, preferred_element_type=jnp.float32
        ) * w

        @pl.when(e == 0)
        def _():
            out_ref[sl, :] = contrib

        @pl.when(e != 0)
        def _():
            out_ref[sl, :] += contrib

        return 0

    jax.lax.fori_loop(0, T // CHUNK, chunk, 0, unroll=False)


@jax.jit
def kernel(x, router_logits_up, router_logits_gate, router_logits_down, w_up, w_gate, w_down):
    del router_logits_gate, router_logits_down
    return pl.pallas_call(
        _moe_body,
        grid=(E,),
        in_specs=[
            pl.BlockSpec((T, E), lambda e: (0, 0)),
            pl.BlockSpec((T, D), lambda e: (0, 0)),
            pl.BlockSpec((1, FF, D), lambda e: (e, 0, 0)),
            pl.BlockSpec((1, FF, D), lambda e: (e, 0, 0)),
            pl.BlockSpec((1, D, FF), lambda e: (e, 0, 0)),
        ],
        out_specs=pl.BlockSpec((T, D), lambda e: (0, 0)),
        out_shape=jax.ShapeDtypeStruct((T, D), jnp.float32),
    )(router_logits_up, x, w_up, w_gate, w_down)
